# baseline (speedup 1.0000x reference)
import functools

import jax
import jax.numpy as jnp
from jax.experimental import pallas as pl
from jax.experimental.pallas import tpu as pltpu
from jax.experimental.pallas import tpu_sc as plsc

N = 10000
E = 160000
D = 256
H = 256
H3 = 768
RB = 1000
NBLK = N // RB
PLANE_ROWS = 5120
HALF = 5000

_PREC = jax.lax.Precision.HIGHEST

EGROUPS = 157
E_PAD = EGROUPS * 1024
EBLK = 1024
DST_PAD = 10008
DEG_ROWS = 10240
Z_ROWS = 10240
HALFE = E_PAD // 2
HCH = 512
def _hist_scs_body(dst_hbm, out_hbm, hist_smem, chunk_smem, sem):
    c = jax.lax.axis_index("core")

    @pl.loop(0, DEG_ROWS)
    def _(i):
        hist_smem[i] = 0.0

    @pl.loop(0, HALFE // HCH)
    def _(ch):
        pltpu.async_copy(
            dst_hbm.at[pl.ds(c * HALFE + ch * HCH, HCH)], chunk_smem,
            sem).wait()

        @pl.loop(0, HCH)
        def _(i):
            d = chunk_smem[i]
            hist_smem[d] += 1.0

    pltpu.async_copy(hist_smem, out_hbm.at[pl.ds(c * DEG_ROWS, DEG_ROWS)],
                     sem).wait()


def _deg_planes_sc(dst_flat):
    krn = pl.kernel(
        _hist_scs_body,
        out_type=jax.ShapeDtypeStruct((2 * DEG_ROWS,), jnp.float32),
        mesh=plsc.ScalarSubcoreMesh(axis_name="core", num_cores=2),
        scratch_types=[
            pltpu.SMEM((DEG_ROWS,), jnp.float32),
            pltpu.SMEM((HCH,), jnp.int32),
            pltpu.SemaphoreType.DMA,
        ],
    )
    return krn(dst_flat).reshape(2, DEG_ROWS, 1)


def _tcscat_kernel(si_ref, di_ref, y_ref, z_ref):
    @pl.when(pl.program_id(0) == 0)
    def _():
        z_ref[...] = jnp.zeros_like(z_ref)

    def body(e, carry):
        s_e = si_ref[0, 0, e]
        d_e = di_ref[0, 0, e]
        z_ref[pl.ds(d_e, 1), :] += y_ref[pl.ds(s_e, 1), :]
        return carry

    jax.lax.fori_loop(0, EBLK, body, 0)


def _z_scatter_tc(s3, d3, y):
    return pl.pallas_call(
        _tcscat_kernel,
        grid=(EGROUPS,),
        in_specs=[
            pl.BlockSpec((1, 1, EBLK), lambda i: (i, 0, 0),
                         memory_space=pltpu.SMEM),
            pl.BlockSpec((1, 1, EBLK), lambda i: (i, 0, 0),
                         memory_space=pltpu.SMEM),
            pl.BlockSpec((N, D), lambda i: (0, 0)),
        ],
        out_specs=pl.BlockSpec((Z_ROWS, D), lambda i: (0, 0)),
        out_shape=jax.ShapeDtypeStruct((Z_ROWS, D), jnp.float32),
    )(s3, d3, y)


def _scale_kernel(deg_ref, x_ref, w_ref, y_ref, dinv_ref):
    deg = deg_ref[0] + deg_ref[1] + 1.0
    dinv = jax.lax.rsqrt(deg)
    xw = jax.lax.dot_general(x_ref[...], w_ref[...], (((1,), (1,)), ((), ())),
                             preferred_element_type=jnp.float32,
                             precision=_PREC)
    y_ref[...] = dinv * xw
    dinv_ref[...] = dinv


def _gi_kernel(z_ref, y_ref, dinv_ref, bg_ref, wt_ref, bias_ref, gi_ref):
    g = dinv_ref[...] * (z_ref[...] + y_ref[...]) + bg_ref[...]
    gi_ref[...] = jax.lax.dot_general(g, wt_ref[...], (((1,), (0,)), ((), ())),
                                      preferred_element_type=jnp.float32,
                                      precision=_PREC) + bias_ref[...]


def _gru_kernel(h0_ref, gi_ref, wt_ref, bhn_ref, out_ref, h_scr):
    @pl.when(pl.program_id(0) == 0)
    def _():
        h_scr[...] = h0_ref[...]

    def body(t, h):
        gih = gi_ref[pl.ds(t, 1), :]
        gh = jax.lax.dot_general(h.astype(jnp.bfloat16), wt_ref[...],
                                 (((1,), (0,)), ((), ())),
                                 preferred_element_type=jnp.float32)
        rz = jax.nn.sigmoid(gih[:, 0:2 * H] + gh[:, 0:2 * H])
        r = rz[:, 0:H]
        zg = rz[:, H:2 * H]
        n = jnp.tanh(gih[:, 2 * H:] + r * (gh[:, 2 * H:] + bhn_ref[...]))
        hn = zg * (h - n) + n
        out_ref[pl.ds(t, 1), :] = hn
        return hn

    h_scr[...] = jax.lax.fori_loop(0, RB, body, h_scr[...])


def kernel(x, edge_index, W_gcn, b_gcn, W_ih, W_hh, b_ih, b_hh, h0):
    pad_src = jnp.zeros((E_PAD - E,), jnp.int32)
    pad_dst = jnp.full((E_PAD - E,), DST_PAD, jnp.int32)
    src_flat = jnp.concatenate([edge_index[0], pad_src])
    dst_flat = jnp.concatenate([edge_index[1], pad_dst])
    s3 = src_flat.reshape(EGROUPS, 1, EBLK)
    d3 = dst_flat.reshape(EGROUPS, 1, EBLK)

    deg_planes = _deg_planes_sc(dst_flat)

    y, dinv = pl.pallas_call(
        _scale_kernel,
        grid=(NBLK,),
        in_specs=[
            pl.BlockSpec((2, RB, 1), lambda i: (0, i, 0)),
            pl.BlockSpec((RB, D), lambda i: (i, 0)),
            pl.BlockSpec((D, D), lambda i: (0, 0)),
        ],
        out_specs=[
            pl.BlockSpec((RB, D), lambda i: (i, 0)),
            pl.BlockSpec((RB, 1), lambda i: (i, 0)),
        ],
        out_shape=[
            jax.ShapeDtypeStruct((N, D), jnp.float32),
            jax.ShapeDtypeStruct((N, 1), jnp.float32),
        ],
    )(deg_planes, x, W_gcn)

    z = _z_scatter_tc(s3, d3, y)

    bias1 = (b_ih + jnp.concatenate([b_hh[:2 * H], jnp.zeros((H,), jnp.float32)]))[None]
    gi = pl.pallas_call(
        _gi_kernel,
        grid=(NBLK,),
        in_specs=[
            pl.BlockSpec((RB, D), lambda i: (i, 0)),
            pl.BlockSpec((RB, D), lambda i: (i, 0)),
            pl.BlockSpec((RB, 1), lambda i: (i, 0)),
            pl.BlockSpec((1, D), lambda i: (0, 0)),
            pl.BlockSpec((D, H3), lambda i: (0, 0)),
            pl.BlockSpec((1, H3), lambda i: (0, 0)),
        ],
        out_specs=pl.BlockSpec((RB, H3), lambda i: (i, 0)),
        out_shape=jax.ShapeDtypeStruct((N, H3), jnp.float32),
    )(z, y, dinv, b_gcn[None], W_ih.T, bias1)

    seq = pl.pallas_call(
        _gru_kernel,
        grid=(NBLK,),
        in_specs=[
            pl.BlockSpec((1, H), lambda i: (0, 0)),
            pl.BlockSpec((RB, H3), lambda i: (i, 0)),
            pl.BlockSpec((H, H3), lambda i: (0, 0)),
            pl.BlockSpec((1, H), lambda i: (0, 0)),
        ],
        out_specs=pl.BlockSpec((RB, H), lambda i: (i, 0)),
        out_shape=jax.ShapeDtypeStruct((N, H), jnp.float32),
        scratch_shapes=[pltpu.VMEM((1, H), jnp.float32)],
    )(h0[0], gi, W_hh.T.astype(jnp.bfloat16), b_hh[2 * H:][None])

    out = seq[None]
    h_n = seq[N - 1:N][None]
    return out, h_n


# speedup vs baseline: 1.3009x; 1.3009x over previous
import functools

import jax
import jax.numpy as jnp
from jax.experimental import pallas as pl
from jax.experimental.pallas import tpu as pltpu
from jax.experimental.pallas import tpu_sc as plsc

N = 10000
E = 160000
D = 256
H = 256
H3 = 768
RB = 1000
NBLK = N // RB
PLANE_ROWS = 5120
HALF = 5000

_PREC = jax.lax.Precision.HIGHEST

EGROUPS = 157
E_PAD = EGROUPS * 1024
EBLK = 1024
DST_PAD = 10008
DEG_ROWS = 10240
Z_ROWS = 10240
HALFE = E_PAD // 2
HCH = 512
def _hist_scs_body(dst_hbm, out_hbm, hist_smem, chunk_smem, sem):
    c = jax.lax.axis_index("core")

    @pl.loop(0, DEG_ROWS)
    def _(i):
        hist_smem[i] = 0.0

    @pl.loop(0, HALFE // HCH)
    def _(ch):
        pltpu.async_copy(
            dst_hbm.at[pl.ds(c * HALFE + ch * HCH, HCH)], chunk_smem,
            sem).wait()

        @pl.loop(0, HCH)
        def _(i):
            d = chunk_smem[i]
            hist_smem[d] += 1.0

    pltpu.async_copy(hist_smem, out_hbm.at[pl.ds(c * DEG_ROWS, DEG_ROWS)],
                     sem).wait()


def _deg_planes_sc(dst_flat):
    krn = pl.kernel(
        _hist_scs_body,
        out_type=jax.ShapeDtypeStruct((2 * DEG_ROWS,), jnp.float32),
        mesh=plsc.ScalarSubcoreMesh(axis_name="core", num_cores=2),
        scratch_types=[
            pltpu.SMEM((DEG_ROWS,), jnp.float32),
            pltpu.SMEM((HCH,), jnp.int32),
            pltpu.SemaphoreType.DMA,
        ],
    )
    return krn(dst_flat).reshape(2, DEG_ROWS, 1)


def _tcscat_kernel(si_ref, di_ref, y_ref, z_ref):
    @pl.when(pl.program_id(0) == 0)
    def _():
        z_ref[...] = jnp.zeros_like(z_ref)

    def body(e, carry):
        s_e = si_ref[0, 0, e]
        d_e = di_ref[0, 0, e]
        z_ref[pl.ds(d_e, 1), :] += y_ref[pl.ds(s_e, 1), :]
        return carry

    jax.lax.fori_loop(0, EBLK, body, 0, unroll=8)


def _z_scatter_tc(s3, d3, y):
    return pl.pallas_call(
        _tcscat_kernel,
        grid=(EGROUPS,),
        in_specs=[
            pl.BlockSpec((1, 1, EBLK), lambda i: (i, 0, 0),
                         memory_space=pltpu.SMEM),
            pl.BlockSpec((1, 1, EBLK), lambda i: (i, 0, 0),
                         memory_space=pltpu.SMEM),
            pl.BlockSpec((N, D), lambda i: (0, 0)),
        ],
        out_specs=pl.BlockSpec((Z_ROWS, D), lambda i: (0, 0)),
        out_shape=jax.ShapeDtypeStruct((Z_ROWS, D), jnp.float32),
    )(s3, d3, y)


def _scale_kernel(deg_ref, x_ref, w_ref, y_ref, dinv_ref):
    deg = deg_ref[0] + deg_ref[1] + 1.0
    dinv = jax.lax.rsqrt(deg)
    xw = jax.lax.dot_general(x_ref[...], w_ref[...], (((1,), (1,)), ((), ())),
                             preferred_element_type=jnp.float32,
                             precision=_PREC)
    y_ref[...] = dinv * xw
    dinv_ref[...] = dinv


def _gi_kernel(z_ref, y_ref, dinv_ref, bg_ref, wt_ref, bias_ref, gi_ref):
    g = dinv_ref[...] * (z_ref[...] + y_ref[...]) + bg_ref[...]
    gi_ref[...] = jax.lax.dot_general(g, wt_ref[...], (((1,), (0,)), ((), ())),
                                      preferred_element_type=jnp.float32,
                                      precision=_PREC) + bias_ref[...]


def _gru_kernel(h0_ref, gi_ref, wt_ref, bhn_ref, out_ref, h_scr):
    @pl.when(pl.program_id(0) == 0)
    def _():
        h_scr[...] = h0_ref[...]

    def body(t, h):
        gih = gi_ref[pl.ds(t, 1), :]
        gh = jax.lax.dot_general(h.astype(jnp.bfloat16), wt_ref[...],
                                 (((1,), (0,)), ((), ())),
                                 preferred_element_type=jnp.float32)
        rz = jax.nn.sigmoid(gih[:, 0:2 * H] + gh[:, 0:2 * H])
        r = rz[:, 0:H]
        zg = rz[:, H:2 * H]
        n = jnp.tanh(gih[:, 2 * H:] + r * (gh[:, 2 * H:] + bhn_ref[...]))
        hn = zg * (h - n) + n
        out_ref[pl.ds(t, 1), :] = hn
        return hn

    h_scr[...] = jax.lax.fori_loop(0, RB, body, h_scr[...], unroll=2)


def kernel(x, edge_index, W_gcn, b_gcn, W_ih, W_hh, b_ih, b_hh, h0):
    pad_src = jnp.zeros((E_PAD - E,), jnp.int32)
    pad_dst = jnp.full((E_PAD - E,), DST_PAD, jnp.int32)
    src_flat = jnp.concatenate([edge_index[0], pad_src])
    dst_flat = jnp.concatenate([edge_index[1], pad_dst])
    s3 = src_flat.reshape(EGROUPS, 1, EBLK)
    d3 = dst_flat.reshape(EGROUPS, 1, EBLK)

    deg_planes = _deg_planes_sc(dst_flat)

    y, dinv = pl.pallas_call(
        _scale_kernel,
        grid=(NBLK,),
        in_specs=[
            pl.BlockSpec((2, RB, 1), lambda i: (0, i, 0)),
            pl.BlockSpec((RB, D), lambda i: (i, 0)),
            pl.BlockSpec((D, D), lambda i: (0, 0)),
        ],
        out_specs=[
            pl.BlockSpec((RB, D), lambda i: (i, 0)),
            pl.BlockSpec((RB, 1), lambda i: (i, 0)),
        ],
        out_shape=[
            jax.ShapeDtypeStruct((N, D), jnp.float32),
            jax.ShapeDtypeStruct((N, 1), jnp.float32),
        ],
    )(deg_planes, x, W_gcn)

    z = _z_scatter_tc(s3, d3, y)

    bias1 = (b_ih + jnp.concatenate([b_hh[:2 * H], jnp.zeros((H,), jnp.float32)]))[None]
    gi = pl.pallas_call(
        _gi_kernel,
        grid=(NBLK,),
        in_specs=[
            pl.BlockSpec((RB, D), lambda i: (i, 0)),
            pl.BlockSpec((RB, D), lambda i: (i, 0)),
            pl.BlockSpec((RB, 1), lambda i: (i, 0)),
            pl.BlockSpec((1, D), lambda i: (0, 0)),
            pl.BlockSpec((D, H3), lambda i: (0, 0)),
            pl.BlockSpec((1, H3), lambda i: (0, 0)),
        ],
        out_specs=pl.BlockSpec((RB, H3), lambda i: (i, 0)),
        out_shape=jax.ShapeDtypeStruct((N, H3), jnp.float32),
    )(z, y, dinv, b_gcn[None], W_ih.T, bias1)

    seq = pl.pallas_call(
        _gru_kernel,
        grid=(NBLK,),
        in_specs=[
            pl.BlockSpec((1, H), lambda i: (0, 0)),
            pl.BlockSpec((RB, H3), lambda i: (i, 0)),
            pl.BlockSpec((H, H3), lambda i: (0, 0)),
            pl.BlockSpec((1, H), lambda i: (0, 0)),
        ],
        out_specs=pl.BlockSpec((RB, H), lambda i: (i, 0)),
        out_shape=jax.ShapeDtypeStruct((N, H), jnp.float32),
        scratch_shapes=[pltpu.VMEM((1, H), jnp.float32)],
    )(h0[0], gi, W_hh.T.astype(jnp.bfloat16), b_hh[2 * H:][None])

    out = seq[None]
    h_n = seq[N - 1:N][None]
    return out, h_n


# speedup vs baseline: 1.3160x; 1.0116x over previous
import functools

import jax
import jax.numpy as jnp
from jax.experimental import pallas as pl
from jax.experimental.pallas import tpu as pltpu
from jax.experimental.pallas import tpu_sc as plsc

N = 10000
E = 160000
D = 256
H = 256
H3 = 768
RB = 1000
NBLK = N // RB
PLANE_ROWS = 5120
HALF = 5000

_PREC = jax.lax.Precision.HIGHEST

EGROUPS = 157
E_PAD = EGROUPS * 1024
EBLK = 1024
DST_PAD = 10008
DEG_ROWS = 10240
Z_ROWS = 10240
HALFE = E_PAD // 2
HCH = 512
def _hist_scs_body(dst_hbm, out_hbm, hist_smem, chunk_smem, sem):
    c = jax.lax.axis_index("core")

    @pl.loop(0, DEG_ROWS)
    def _(i):
        hist_smem[i] = 0.0

    @pl.loop(0, HALFE // HCH)
    def _(ch):
        pltpu.async_copy(
            dst_hbm.at[pl.ds(c * HALFE + ch * HCH, HCH)], chunk_smem,
            sem).wait()

        @pl.loop(0, HCH)
        def _(i):
            d = chunk_smem[i]
            hist_smem[d] += 1.0

    pltpu.async_copy(hist_smem, out_hbm.at[pl.ds(c * DEG_ROWS, DEG_ROWS)],
                     sem).wait()


def _deg_planes_sc(dst_flat):
    krn = pl.kernel(
        _hist_scs_body,
        out_type=jax.ShapeDtypeStruct((2 * DEG_ROWS,), jnp.float32),
        mesh=plsc.ScalarSubcoreMesh(axis_name="core", num_cores=2),
        scratch_types=[
            pltpu.SMEM((DEG_ROWS,), jnp.float32),
            pltpu.SMEM((HCH,), jnp.int32),
            pltpu.SemaphoreType.DMA,
        ],
    )
    return krn(dst_flat).reshape(2, DEG_ROWS, 1)


def _tcscat_kernel(si_ref, di_ref, y_ref, z_ref):
    @pl.when(pl.program_id(0) == 0)
    def _():
        z_ref[...] = jnp.zeros_like(z_ref)

    def body(e, carry):
        s_e = si_ref[0, 0, e]
        d_e = di_ref[0, 0, e]
        z_ref[pl.ds(d_e, 1), :] += y_ref[pl.ds(s_e, 1), :]
        return carry

    jax.lax.fori_loop(0, EBLK, body, 0, unroll=16)


def _z_scatter_tc(s3, d3, y):
    return pl.pallas_call(
        _tcscat_kernel,
        grid=(EGROUPS,),
        in_specs=[
            pl.BlockSpec((1, 1, EBLK), lambda i: (i, 0, 0),
                         memory_space=pltpu.SMEM),
            pl.BlockSpec((1, 1, EBLK), lambda i: (i, 0, 0),
                         memory_space=pltpu.SMEM),
            pl.BlockSpec((N, D), lambda i: (0, 0)),
        ],
        out_specs=pl.BlockSpec((Z_ROWS, D), lambda i: (0, 0)),
        out_shape=jax.ShapeDtypeStruct((Z_ROWS, D), jnp.float32),
    )(s3, d3, y)


def _scale_kernel(deg_ref, x_ref, w_ref, y_ref, dinv_ref):
    deg = deg_ref[0] + deg_ref[1] + 1.0
    dinv = jax.lax.rsqrt(deg)
    xw = jax.lax.dot_general(x_ref[...], w_ref[...], (((1,), (1,)), ((), ())),
                             preferred_element_type=jnp.float32,
                             precision=_PREC)
    y_ref[...] = dinv * xw
    dinv_ref[...] = dinv


def _gi_kernel(z_ref, y_ref, dinv_ref, bg_ref, wt_ref, bias_ref, gi_ref):
    g = dinv_ref[...] * (z_ref[...] + y_ref[...]) + bg_ref[...]
    gi_ref[...] = jax.lax.dot_general(g, wt_ref[...], (((1,), (0,)), ((), ())),
                                      preferred_element_type=jnp.float32,
                                      precision=_PREC) + bias_ref[...]


def _gru_kernel(h0_ref, gi_ref, wt_ref, bhn_ref, out_ref, h_scr):
    @pl.when(pl.program_id(0) == 0)
    def _():
        h_scr[...] = h0_ref[...]

    def body(t, h):
        gih = gi_ref[pl.ds(t, 1), :]
        gh = jax.lax.dot_general(h.astype(jnp.bfloat16), wt_ref[...],
                                 (((1,), (0,)), ((), ())),
                                 preferred_element_type=jnp.float32)
        rz = jax.nn.sigmoid(gih[:, 0:2 * H] + gh[:, 0:2 * H])
        r = rz[:, 0:H]
        zg = rz[:, H:2 * H]
        n = jnp.tanh(gih[:, 2 * H:] + r * (gh[:, 2 * H:] + bhn_ref[...]))
        hn = zg * (h - n) + n
        out_ref[pl.ds(t, 1), :] = hn
        return hn

    h_scr[...] = jax.lax.fori_loop(0, RB, body, h_scr[...], unroll=4)


def kernel(x, edge_index, W_gcn, b_gcn, W_ih, W_hh, b_ih, b_hh, h0):
    pad_src = jnp.zeros((E_PAD - E,), jnp.int32)
    pad_dst = jnp.full((E_PAD - E,), DST_PAD, jnp.int32)
    src_flat = jnp.concatenate([edge_index[0], pad_src])
    dst_flat = jnp.concatenate([edge_index[1], pad_dst])
    s3 = src_flat.reshape(EGROUPS, 1, EBLK)
    d3 = dst_flat.reshape(EGROUPS, 1, EBLK)

    deg_planes = _deg_planes_sc(dst_flat)

    y, dinv = pl.pallas_call(
        _scale_kernel,
        grid=(NBLK,),
        in_specs=[
            pl.BlockSpec((2, RB, 1), lambda i: (0, i, 0)),
            pl.BlockSpec((RB, D), lambda i: (i, 0)),
            pl.BlockSpec((D, D), lambda i: (0, 0)),
        ],
        out_specs=[
            pl.BlockSpec((RB, D), lambda i: (i, 0)),
            pl.BlockSpec((RB, 1), lambda i: (i, 0)),
        ],
        out_shape=[
            jax.ShapeDtypeStruct((N, D), jnp.float32),
            jax.ShapeDtypeStruct((N, 1), jnp.float32),
        ],
    )(deg_planes, x, W_gcn)

    z = _z_scatter_tc(s3, d3, y)

    bias1 = (b_ih + jnp.concatenate([b_hh[:2 * H], jnp.zeros((H,), jnp.float32)]))[None]
    gi = pl.pallas_call(
        _gi_kernel,
        grid=(NBLK,),
        in_specs=[
            pl.BlockSpec((RB, D), lambda i: (i, 0)),
            pl.BlockSpec((RB, D), lambda i: (i, 0)),
            pl.BlockSpec((RB, 1), lambda i: (i, 0)),
            pl.BlockSpec((1, D), lambda i: (0, 0)),
            pl.BlockSpec((D, H3), lambda i: (0, 0)),
            pl.BlockSpec((1, H3), lambda i: (0, 0)),
        ],
        out_specs=pl.BlockSpec((RB, H3), lambda i: (i, 0)),
        out_shape=jax.ShapeDtypeStruct((N, H3), jnp.float32),
    )(z, y, dinv, b_gcn[None], W_ih.T, bias1)

    seq = pl.pallas_call(
        _gru_kernel,
        grid=(NBLK,),
        in_specs=[
            pl.BlockSpec((1, H), lambda i: (0, 0)),
            pl.BlockSpec((RB, H3), lambda i: (i, 0)),
            pl.BlockSpec((H, H3), lambda i: (0, 0)),
            pl.BlockSpec((1, H), lambda i: (0, 0)),
        ],
        out_specs=pl.BlockSpec((RB, H), lambda i: (i, 0)),
        out_shape=jax.ShapeDtypeStruct((N, H), jnp.float32),
        scratch_shapes=[pltpu.VMEM((1, H), jnp.float32)],
    )(h0[0], gi, W_hh.T.astype(jnp.bfloat16), b_hh[2 * H:][None])

    out = seq[None]
    h_n = seq[N - 1:N][None]
    return out, h_n
